# baseline (speedup 1.0000x reference)
import jax
import jax.numpy as jnp
from jax import lax
from jax.experimental import pallas as pl
from jax.experimental.pallas import tpu as pltpu
from jax.experimental.pallas import tpu_sc as plsc

VOCAB = 8192
BATCH = 4
SEQ = 2048
D = VOCAB
NC, NS = 2, 16
NW = NC * NS
B = BATCH * SEQ
B_PER_W = B // NW
CHUNK = 4
RING = 3
NSTEP = B_PER_W // CHUNK
NIDX = B_PER_W + CHUNK
QPAD = B_PER_W + 16


def _body(table_hbm, uidx_hbm, qrank_hbm, pos_hbm, out_hbm,
          uv, qv, pv, rbufs, wsems):
    wid = lax.axis_index("s") * NC + lax.axis_index("c")
    pltpu.sync_copy(uidx_hbm.at[wid], uv)
    pltpu.sync_copy(qrank_hbm.at[wid], qv)
    pltpu.sync_copy(pos_hbm.at[wid], pv)

    def step(k, q_hi):
        i0 = k * CHUNK
        vq = qv[pl.ds(i0, 16)]
        vp = pv[pl.ds(i0, 16)]
        q3 = vq[CHUNK - 1]
        mstar = q3 // CHUNK

        @pl.when(mstar > q_hi)
        def _():
            m = q_hi + 1
            msel = lax.rem(m, RING)
            for r in range(RING):
                @pl.when(msel == r)
                def _():
                    pltpu.sync_copy(table_hbm.at[uv.at[m]], rbufs[r])

        for j in range(CHUNK):
            qj = vq[j]
            pj = vp[j]

            @pl.when(k > 0)
            def _():
                pltpu.make_async_copy(
                    rbufs[0].at[pl.ds(0, 1)], out_hbm.at[pl.ds(0, 1)],
                    wsems[j],
                ).wait()

            csel = lax.rem(qj // CHUNK, RING)
            lane = lax.rem(qj, CHUNK)
            for r in range(RING):
                @pl.when(csel == r)
                def _():
                    pltpu.async_copy(
                        rbufs[r].at[pl.ds(lane, 1)],
                        out_hbm.at[pl.ds(pj, 1)],
                        wsems[j],
                    )
        return jnp.where(mstar > q_hi, q_hi + 1, q_hi)

    lax.fori_loop(0, NSTEP, step, jnp.int32(-1))

    for j in range(CHUNK):
        pltpu.make_async_copy(
            rbufs[0].at[pl.ds(0, 1)], out_hbm.at[pl.ds(0, 1)], wsems[j]
        ).wait()


@jax.jit
def _sc_gather(uidx, qrank, pos, table):
    mesh = plsc.VectorSubcoreMesh(core_axis_name="c", subcore_axis_name="s")
    scratch = (
        pltpu.VMEM((NIDX // CHUNK, CHUNK), jnp.int32),
        pltpu.VMEM((QPAD,), jnp.int32),
        pltpu.VMEM((QPAD,), jnp.int32),
        tuple(pltpu.VMEM((CHUNK, D), jnp.float32) for _ in range(RING)),
        tuple(pltpu.SemaphoreType.DMA for _ in range(CHUNK)),
    )
    run = pl.kernel(
        _body,
        out_type=jax.ShapeDtypeStruct((B, D), jnp.float32),
        mesh=mesh,
        scratch_types=scratch,
    )
    return run(table, uidx, qrank, pos)


def kernel(context, table):
    flat = context.astype(jnp.int32).ravel()
    order = jnp.argsort(flat)
    sidx = flat[order]
    new = jnp.concatenate(
        [jnp.ones((1,), jnp.int32), (sidx[1:] != sidx[:-1]).astype(jnp.int32)]
    )
    rank = jnp.cumsum(new) - 1
    upad = (jnp.arange(B + NIDX, dtype=jnp.int32) * 37) % VOCAB
    uniq = upad.at[rank].set(sidx)
    rank2 = rank.reshape(NW, B_PER_W)
    r_start = rank2[:, :1]
    qrank = (rank2 - r_start).astype(jnp.int32)
    qrank = jnp.pad(qrank, ((0, 0), (0, QPAD - B_PER_W)), mode="edge")
    uidx = uniq[r_start + jnp.arange(NIDX, dtype=jnp.int32)[None, :]]
    uidx = uidx.reshape(NW, NIDX // CHUNK, CHUNK)
    pos = jnp.pad(order.astype(jnp.int32).reshape(NW, B_PER_W),
                  ((0, 0), (0, QPAD - B_PER_W)), mode="edge")
    out = _sc_gather(uidx, qrank, pos, table)
    return out.reshape(BATCH, SEQ, D)


# speedup vs baseline: 1.2956x; 1.2956x over previous
import jax
import jax.numpy as jnp
from jax import lax
from jax.experimental import pallas as pl
from jax.experimental.pallas import tpu as pltpu
from jax.experimental.pallas import tpu_sc as plsc

VOCAB = 8192
BATCH = 4
SEQ = 2048
D = VOCAB
NC, NS = 2, 16
NW = NC * NS
B = BATCH * SEQ
B_PER_W = B // NW
CHUNK = 4
NBUF = 3
NCHUNK = B_PER_W // CHUNK
NROUND = -(-NCHUNK // NBUF) * NBUF


def _gather_body(table_hbm, ctx_hbm, out_hbm, idx_v, rows, gsems, ssems):
    wid = lax.axis_index("s") * NC + lax.axis_index("c")
    base = wid * B_PER_W
    pltpu.sync_copy(ctx_hbm.at[wid], idx_v)

    def start_gather(g, b):
        return pltpu.async_copy(table_hbm.at[idx_v.at[g]], rows[b], gsems[b])

    def out_slice(g):
        return out_hbm.at[pl.ds(base + g * CHUNK, CHUNK)]

    for b in range(NBUF):
        start_gather(b, b)

    @pl.loop(0, NROUND, step=NBUF)
    def _(g0):
        for b in range(NBUF):
            g = g0 + b

            @pl.when(g < NCHUNK)
            def _():
                pltpu.make_async_copy(
                    table_hbm.at[idx_v.at[g]], rows[b], gsems[b]
                ).wait()
                out_copy = pltpu.async_copy(rows[b], out_slice(g), ssems[b])
                @pl.when(g + NBUF < NCHUNK)
                def _():
                    out_copy.wait()
                    start_gather(g + NBUF, b)

    for b in range(NBUF):
        g = NCHUNK - 1 - ((NCHUNK - 1 - b) % NBUF)
        pltpu.make_async_copy(rows[b], out_slice(g), ssems[b]).wait()


@jax.jit
def _sc_gather(ctx3, table):
    mesh = plsc.VectorSubcoreMesh(core_axis_name="c", subcore_axis_name="s")
    scratch = (
        pltpu.VMEM((NCHUNK, CHUNK), jnp.int32),
        tuple(pltpu.VMEM((CHUNK, D), jnp.float32) for _ in range(NBUF)),
        tuple(pltpu.SemaphoreType.DMA for _ in range(NBUF)),
        tuple(pltpu.SemaphoreType.DMA for _ in range(NBUF)),
    )
    run = pl.kernel(
        _gather_body,
        out_type=jax.ShapeDtypeStruct((B, D), jnp.float32),
        mesh=mesh,
        scratch_types=scratch,
    )
    return run(table, ctx3)


def kernel(context, table):
    ctx3 = context.astype(jnp.int32).reshape(NW, NCHUNK, CHUNK)
    out = _sc_gather(ctx3, table)
    return out.reshape(BATCH, SEQ, D)
